# baseline (speedup 1.0000x reference)
import functools

import jax
import jax.numpy as jnp
from jax import lax
from jax.experimental import pallas as pl
from jax.experimental.pallas import tpu as pltpu
from jax.experimental.pallas import tpu_sc as plsc

N, C, H, W = 4, 96, 224, 224
NHW = N * H * W
CP = 128
R = 10240
NPASS = 10
ROWS_PAD = 204800
NSUB = 16
SHARE = NHW // (2 * NSUB) * 2
SEG = 896
NSEGV = SEG // 16
G = 32
STRIDE = SEG + G
TPS = R // NSUB
NB = 16


def _prep_body(flo_ref, ids_ref, w_ref, smin_ref, smax_ref):
    n = pl.program_id(0)
    y = flo_ref[0, 0]
    x = flo_ref[0, 1]
    x1 = jnp.floor(x)
    y1 = jnp.floor(y)
    fx = x - x1
    fy = y - y1
    ih = lax.broadcasted_iota(jnp.int32, (H, W), 0)
    iw = lax.broadcasted_iota(jnp.int32, (H, W), 1)
    ix1 = x1.astype(jnp.int32) + ih
    iy1 = y1.astype(jnp.int32) + iw
    ix2 = ix1 + 1
    iy2 = iy1 + 1
    base = n * (H * W) + ih * W + iw
    a1 = fx * fx
    a2 = (fx - 1.0) * (fx - 1.0)
    b1 = fy * fy
    b2 = (fy - 1.0) * (fy - 1.0)
    mx1 = (ix1 >= 0) & (ix1 < H)
    mx2 = (ix2 >= 0) & (ix2 < H)
    my1 = (iy1 >= 0) & (iy1 < W)
    my2 = (iy2 >= 0) & (iy2 < W)
    combos = (
        (mx1 & my1, ix1, iy1, jnp.exp(-(a1 + b1))),
        (mx1 & my2, ix1, iy2, jnp.exp(-(a1 + b2))),
        (mx2 & my1, ix2, iy1, jnp.exp(-(a2 + b1))),
        (mx2 & my2, ix2, iy2, jnp.exp(-(a2 + b2))),
    )
    tmin = None
    tmax = None
    for k, (m, ix, iy, wk) in enumerate(combos):
        tid = n * (H * W) + ix * W + iy
        tid = jnp.where(m, tid, base)
        ids_ref[k, 0] = tid
        w_ref[k, 0] = jnp.where(m, wk, 0.0)
        tmin = tid if tmin is None else jnp.minimum(tmin, tid)
        tmax = tid if tmax is None else jnp.maximum(tmax, tid)
    sm = jnp.min(jnp.min(tmin.reshape(56, 4, W), axis=1), axis=1)
    sx = jnp.max(jnp.max(tmax.reshape(56, 4, W), axis=1), axis=1)
    smin_ref[0, 0] = jnp.concatenate([sm, jnp.full((8,), 2 ** 30, jnp.int32)])
    smax_ref[0, 0] = jnp.concatenate([sx, jnp.full((8,), -1, jnp.int32)])


def _prep(flo):
    return pl.pallas_call(
        _prep_body,
        grid=(N,),
        in_specs=[pl.BlockSpec((1, 2, H, W), lambda n: (n, 0, 0, 0))],
        out_specs=[
            pl.BlockSpec((4, 1, H, W), lambda n: (0, n, 0, 0)),
            pl.BlockSpec((4, 1, H, W), lambda n: (0, n, 0, 0)),
            pl.BlockSpec((1, 8, 64), lambda n: (n, 0, 0)),
            pl.BlockSpec((1, 8, 64), lambda n: (n, 0, 0)),
        ],
        out_shape=[
            jax.ShapeDtypeStruct((4, N, H, W), jnp.int32),
            jax.ShapeDtypeStruct((4, N, H, W), jnp.float32),
            jax.ShapeDtypeStruct((N, 8, 64), jnp.int32),
            jax.ShapeDtypeStruct((N, 8, 64), jnp.int32),
        ],
    )(flo)


_HC = 7
_HB = H // _HC
_RB = _HB * W


def _tin_body(img_ref, out_ref):
    x = img_ref[0].reshape(C, _RB)
    xp = jnp.concatenate([x, jnp.zeros((CP - C, _RB), jnp.float32)], axis=0)
    out_ref[...] = xp.T


def _to_rows(img):
    return pl.pallas_call(
        _tin_body,
        grid=(N, _HC),
        in_specs=[pl.BlockSpec((1, C, _HB, W), lambda n, h: (n, 0, h, 0))],
        out_specs=pl.BlockSpec((_RB, CP), lambda n, h: (n * _HC + h, 0)),
        out_shape=jax.ShapeDtypeStruct((NHW, CP), jnp.float32),
    )(img)


def _post_body(rows_ref, ws_ref, img_ref, one_ref):
    xt = rows_ref[...].T
    img_ref[0] = xt[:C].reshape(C, _HB, W)
    wv = ws_ref[...]
    one_ref[0] = jnp.broadcast_to(wv[None, :], (C, _RB)).reshape(C, _HB, W)


def _post(rows, wsum):
    return pl.pallas_call(
        _post_body,
        grid=(N, _HC),
        in_specs=[
            pl.BlockSpec((_RB, CP), lambda n, h: (n * _HC + h, 0)),
            pl.BlockSpec((_RB,), lambda n, h: (n * _HC + h,)),
        ],
        out_specs=[
            pl.BlockSpec((1, C, _HB, W), lambda n, h: (n, 0, h, 0)),
            pl.BlockSpec((1, C, _HB, W), lambda n, h: (n, 0, h, 0)),
        ],
        out_shape=[
            jax.ShapeDtypeStruct((N, C, H, W), jnp.float32),
            jax.ShapeDtypeStruct((N, C, H, W), jnp.float32),
        ],
    )(rows, wsum)


def _sc_body(imgt, ids4, w4, smin, smax, out_rows, out_ws,
             stg_i, stg_w, src_b, idx_b, wv_b, rows_v, scaled, idx_c,
             zb, zw, vb, wb, sm_v, sx_v, band, wband,
             sem_t, sem_g, sem_s, sem_z, sem_w):
    cid = lax.axis_index("c")
    sid = lax.axis_index("s")
    opq = (cid + sid) >> 10
    iota = lax.iota(jnp.int32, 16)
    pad_src = (iota * 401 + (sid * 2 + cid) * 977) % NHW
    pad_idx = (iota * 73 + sid * 131) % R
    zeros16 = jnp.zeros((16,), jnp.float32)

    def _z1(t, _):
        i, j = t // 8, t % 8
        zb[i, pl.ds(j * 16, 16)] = zeros16
        return 0
    lax.fori_loop(0, NB * 8, _z1, 0)

    def _z2(j, _):
        zw[pl.ds(j * 16, 16)] = zeros16
        return 0
    lax.fori_loop(0, TPS // 16, _z2, 0)

    pltpu.async_copy(smin, sm_v, sem_g).wait()
    pltpu.async_copy(smax, sx_v, sem_g).wait()

    def _zero_shared():
        def _zf(j, _):
            pltpu.async_copy(
                zb, band.at[pl.ds(sid * TPS + j * NB, NB)], sem_z)
            return 0

        def _zd(j, _):
            pltpu.make_async_copy(
                zb, band.at[pl.ds(sid * TPS + j * NB, NB)], sem_z).wait()
            return 0

        lax.fori_loop(0, TPS // NB + opq, _zf, 0)
        pltpu.async_copy(zw, wband.at[pl.ds(sid * TPS, TPS)], sem_z)
        lax.fori_loop(0, TPS // NB + opq, _zd, 0)
        pltpu.make_async_copy(
            zw, wband.at[pl.ds(sid * TPS, TPS)], sem_z).wait()

    with jax.named_scope("zero0"):
        _zero_shared()
    plsc.subcore_barrier()

    def _pass(p, _):
        lo = (2 * p + cid) * R
        hi = lo + R

        def _segment(seg, _):
            seg_off = sid * SHARE + seg * SEG
            gs = (sid // 4) * 64 + (sid % 4) * 14 + seg
            gsv = jnp.full((16,), gs, jnp.int32)
            sm = plsc.load_gather(sm_v, [gsv])[0]
            sx = plsc.load_gather(sx_v, [gsv])[0]

            @pl.when((sx >= lo) & (sm < hi))
            def _do_segment():
                def _stf(k, _):
                    pltpu.async_copy(ids4.at[k, pl.ds(seg_off, SEG)],
                                     stg_i.at[k], sem_t)
                    pltpu.async_copy(w4.at[k, pl.ds(seg_off, SEG)],
                                     stg_w.at[k], sem_t)
                    return 0

                def _std(k, _):
                    pltpu.make_async_copy(ids4.at[k, pl.ds(seg_off, SEG)],
                                          stg_i.at[k], sem_t).wait()
                    pltpu.make_async_copy(w4.at[k, pl.ds(seg_off, SEG)],
                                          stg_w.at[k], sem_t).wait()
                    return 0

                with jax.named_scope("stage_in"):
                    lax.fori_loop(0, 4 + opq, _stf, 0)
                    lax.fori_loop(0, 4 + opq, _std, 0)

                def _scan(v, count):
                    tids = [stg_i[k, pl.ds(v * 16, 16)] for k in range(4)]
                    inb = [(t >= lo) & (t < hi) for t in tids]
                    m = inb[0] | inb[1] | inb[2] | inb[3]
                    cnt = plsc.all_reduce_population_count(m)[0]
                    svec = iota + (seg_off + v * 16)
                    plsc.store_compressed(src_b.at[pl.ds(count, 16)], svec,
                                          mask=m)
                    for k in range(4):
                        lidx = jnp.clip(tids[k] - lo, 0, R - 1)
                        wk = stg_w[k, pl.ds(v * 16, 16)]
                        weff = jnp.where(inb[k], wk, 0.0)
                        plsc.store_compressed(
                            idx_b.at[pl.ds(k * STRIDE + count, 16)], lidx,
                            mask=m)
                        plsc.store_compressed(
                            wv_b.at[pl.ds(k * STRIDE + count, 16)], weff,
                            mask=m)
                    return count + cnt

                with jax.named_scope("scan"):
                    count = lax.fori_loop(0, NSEGV, _scan, 0)

                target = (count + G - 1) // G * G

                def _pad(i, c):
                    src_b[pl.ds(c, 16)] = pad_src
                    for k in range(4):
                        idx_b[pl.ds(k * STRIDE + c, 16)] = pad_idx
                        wv_b[pl.ds(k * STRIDE + c, 16)] = zeros16
                    return c + 16

                lax.fori_loop(0, (target - count + 15) // 16, _pad, count)

                def _drain_sc():
                    def _d(k, _):
                        pltpu.make_async_copy(
                            scaled.at[k], band.at[idx_c.at[k]],
                            sem_s).wait()
                        pltpu.make_async_copy(
                            wv_b.at[pl.ds(k * STRIDE, G)],
                            wband.at[idx_c.at[k]], sem_s).wait()
                        return 0

                    lax.fori_loop(0, 4 + opq, _d, 0)

                def _chunk(ch, _):
                    c0 = ch * G
                    gat = pltpu.async_copy(
                        imgt.at[src_b.at[pl.ds(c0, G)]], rows_v, sem_g)

                    @pl.when(ch > 0)
                    def _():
                        _drain_sc()

                    for k in range(4):
                        for j in range(G // 16):
                            idx_c[k, pl.ds(j * 16, 16)] = (
                                idx_b[pl.ds(k * STRIDE + c0 + j * 16, 16)])
                    gat.wait()

                    def _row(r, _):
                        rv = [rows_v[r, pl.ds(i * 16, 16)]
                              for i in range(8)]
                        for k in range(4):
                            ridx = jnp.full(
                                (16,), k * STRIDE + c0 + r, jnp.int32)
                            wsc = plsc.load_gather(wv_b, [ridx])
                            for i in range(8):
                                scaled[k, r, pl.ds(i * 16, 16)] = (
                                    rv[i] * wsc)
                        return 0

                    lax.fori_loop(0, G, _row, 0)

                    def _f(k, _):
                        pltpu.async_copy(
                            scaled.at[k], band.at[idx_c.at[k]], sem_s,
                            add=True)
                        pltpu.async_copy(
                            wv_b.at[pl.ds(k * STRIDE + c0, G)],
                            wband.at[idx_c.at[k]], sem_s, add=True)
                        return 0

                    lax.fori_loop(0, 4 + opq, _f, 0)
                    return 0

                with jax.named_scope("chunks"):
                    lax.fori_loop(0, target // G, _chunk, 0)

                @pl.when(target > 0)
                def _():
                    _drain_sc()

            return 0

        lax.fori_loop(0, SHARE // SEG, _segment, 0)
        plsc.subcore_barrier()
        g0 = (2 * p + cid) * R + sid * TPS

        def _wo(j, _):
            b = j % 2

            @pl.when(j >= 2)
            def _():
                pltpu.make_async_copy(
                    vb.at[b], out_rows.at[pl.ds(g0 + j * NB, NB)],
                    sem_w).wait()

            pltpu.async_copy(band.at[pl.ds(sid * TPS + j * NB, NB)],
                             vb.at[b], sem_g).wait()
            pltpu.async_copy(vb.at[b], out_rows.at[pl.ds(g0 + j * NB, NB)],
                             sem_w)
            return 0

        with jax.named_scope("writeout"):
            lax.fori_loop(0, TPS // NB + opq, _wo, 0)

        def _wd(j, _):
            pltpu.make_async_copy(
                vb.at[j], out_rows.at[pl.ds(g0 + j * NB, NB)], sem_w).wait()
            return 0

        lax.fori_loop(0, 2 + opq, _wd, 0)
        pltpu.async_copy(wband.at[pl.ds(sid * TPS, TPS)], wb, sem_g).wait()
        pltpu.async_copy(wb, out_ws.at[pl.ds(g0, TPS)], sem_g).wait()
        with jax.named_scope("rezero"):
            _zero_shared()
        plsc.subcore_barrier()
        return 0

    lax.fori_loop(0, NPASS, _pass, 0)


def _sc_scatter(imgt, ids4, w4, smin, smax):
    kfn = pl.kernel(
        _sc_body,
        out_type=[
            jax.ShapeDtypeStruct((ROWS_PAD, CP), jnp.float32),
            jax.ShapeDtypeStruct((ROWS_PAD,), jnp.float32),
        ],
        mesh=plsc.VectorSubcoreMesh(core_axis_name="c", subcore_axis_name="s"),
        compiler_params=pltpu.CompilerParams(needs_layout_passes=False),
        scratch_types=[
            pltpu.VMEM((4, SEG), jnp.int32),
            pltpu.VMEM((4, SEG), jnp.float32),
            pltpu.VMEM((SEG + G,), jnp.int32),
            pltpu.VMEM((4 * (SEG + G),), jnp.int32),
            pltpu.VMEM((4 * (SEG + G),), jnp.float32),
            pltpu.VMEM((G, CP), jnp.float32),
            pltpu.VMEM((4, G, CP), jnp.float32),
            pltpu.VMEM((4, G), jnp.int32),
            pltpu.VMEM((NB, CP), jnp.float32),
            pltpu.VMEM((TPS,), jnp.float32),
            pltpu.VMEM((2, NB, CP), jnp.float32),
            pltpu.VMEM((TPS,), jnp.float32),
            pltpu.VMEM((256,), jnp.int32),
            pltpu.VMEM((256,), jnp.int32),
            pltpu.VMEM_SHARED((R, CP), jnp.float32),
            pltpu.VMEM_SHARED((R,), jnp.float32),
            pltpu.SemaphoreType.DMA,
            pltpu.SemaphoreType.DMA,
            pltpu.SemaphoreType.DMA,
            pltpu.SemaphoreType.DMA,
            pltpu.SemaphoreType.DMA,
        ],
    )
    return kfn(imgt, ids4, w4, smin, smax)


def kernel(img, flo):
    ids4, w4, smin, smax = _prep(flo)
    ids4 = ids4.reshape(4, NHW)
    w4 = w4.reshape(4, NHW)
    smin = smin[:, 0, :].reshape(256)
    smax = smax[:, 0, :].reshape(256)
    imgt = _to_rows(img)
    out_rows, wsum = _sc_scatter(imgt, ids4, w4, smin, smax)
    return _post(out_rows[:NHW], wsum[:NHW])


# speedup vs baseline: 5.9548x; 5.9548x over previous
import functools

import jax
import jax.numpy as jnp
from jax import lax
from jax.experimental import pallas as pl
from jax.experimental.pallas import tpu as pltpu
from jax.experimental.pallas import tpu_sc as plsc

N, C, H, W = 4, 96, 224, 224
NHW = N * H * W
CP = 128
R = 10240
NPASS = 10
ROWS_PAD = 204800
NSUB = 16
SHARE = NHW // (2 * NSUB) * 2
SEG = 896
NSEGV = SEG // 16
G = 32
STRIDE = SEG + G
TPS = R // NSUB
NB = 16


def _prep_body(flo_ref, ids_ref, w_ref, smin_ref, smax_ref):
    n = pl.program_id(0)
    y = flo_ref[0, 0]
    x = flo_ref[0, 1]
    x1 = jnp.floor(x)
    y1 = jnp.floor(y)
    fx = x - x1
    fy = y - y1
    ih = lax.broadcasted_iota(jnp.int32, (H, W), 0)
    iw = lax.broadcasted_iota(jnp.int32, (H, W), 1)
    ix1 = x1.astype(jnp.int32) + ih
    iy1 = y1.astype(jnp.int32) + iw
    ix2 = ix1 + 1
    iy2 = iy1 + 1
    base = n * (H * W) + ih * W + iw
    a1 = fx * fx
    a2 = (fx - 1.0) * (fx - 1.0)
    b1 = fy * fy
    b2 = (fy - 1.0) * (fy - 1.0)
    mx1 = (ix1 >= 0) & (ix1 < H)
    mx2 = (ix2 >= 0) & (ix2 < H)
    my1 = (iy1 >= 0) & (iy1 < W)
    my2 = (iy2 >= 0) & (iy2 < W)
    combos = (
        (mx1 & my1, ix1, iy1, jnp.exp(-(a1 + b1))),
        (mx1 & my2, ix1, iy2, jnp.exp(-(a1 + b2))),
        (mx2 & my1, ix2, iy1, jnp.exp(-(a2 + b1))),
        (mx2 & my2, ix2, iy2, jnp.exp(-(a2 + b2))),
    )
    tmin = None
    tmax = None
    for k, (m, ix, iy, wk) in enumerate(combos):
        tid = n * (H * W) + ix * W + iy
        tid = jnp.where(m, tid, base)
        ids_ref[k, 0] = tid
        w_ref[k, 0] = jnp.where(m, wk, 0.0)
        tmin = tid if tmin is None else jnp.minimum(tmin, tid)
        tmax = tid if tmax is None else jnp.maximum(tmax, tid)
    sm = jnp.min(jnp.min(tmin.reshape(56, 4, W), axis=1), axis=1)
    sx = jnp.max(jnp.max(tmax.reshape(56, 4, W), axis=1), axis=1)
    smin_ref[0, 0] = jnp.concatenate([sm, jnp.full((8,), 2 ** 30, jnp.int32)])
    smax_ref[0, 0] = jnp.concatenate([sx, jnp.full((8,), -1, jnp.int32)])


def _prep(flo):
    return pl.pallas_call(
        _prep_body,
        grid=(N,),
        in_specs=[pl.BlockSpec((1, 2, H, W), lambda n: (n, 0, 0, 0))],
        out_specs=[
            pl.BlockSpec((4, 1, H, W), lambda n: (0, n, 0, 0)),
            pl.BlockSpec((4, 1, H, W), lambda n: (0, n, 0, 0)),
            pl.BlockSpec((1, 8, 64), lambda n: (n, 0, 0)),
            pl.BlockSpec((1, 8, 64), lambda n: (n, 0, 0)),
        ],
        out_shape=[
            jax.ShapeDtypeStruct((4, N, H, W), jnp.int32),
            jax.ShapeDtypeStruct((4, N, H, W), jnp.float32),
            jax.ShapeDtypeStruct((N, 8, 64), jnp.int32),
            jax.ShapeDtypeStruct((N, 8, 64), jnp.int32),
        ],
    )(flo)


_HC = 7
_HB = H // _HC
_RB = _HB * W


def _tin_body(img_ref, out_ref):
    x = img_ref[0].reshape(C, _RB)
    xp = jnp.concatenate([x, jnp.zeros((CP - C, _RB), jnp.float32)], axis=0)
    out_ref[...] = xp.T


def _to_rows(img):
    return pl.pallas_call(
        _tin_body,
        grid=(N, _HC),
        in_specs=[pl.BlockSpec((1, C, _HB, W), lambda n, h: (n, 0, h, 0))],
        out_specs=pl.BlockSpec((_RB, CP), lambda n, h: (n * _HC + h, 0)),
        out_shape=jax.ShapeDtypeStruct((NHW, CP), jnp.float32),
    )(img)


def _post_body(rows_ref, ws_ref, img_ref, one_ref):
    xt = rows_ref[...].T
    img_ref[0] = xt[:C].reshape(C, _HB, W)
    wv = ws_ref[...]
    one_ref[0] = jnp.broadcast_to(wv[None, :], (C, _RB)).reshape(C, _HB, W)


def _post(rows, wsum):
    return pl.pallas_call(
        _post_body,
        grid=(N, _HC),
        in_specs=[
            pl.BlockSpec((_RB, CP), lambda n, h: (n * _HC + h, 0)),
            pl.BlockSpec((_RB,), lambda n, h: (n * _HC + h,)),
        ],
        out_specs=[
            pl.BlockSpec((1, C, _HB, W), lambda n, h: (n, 0, h, 0)),
            pl.BlockSpec((1, C, _HB, W), lambda n, h: (n, 0, h, 0)),
        ],
        out_shape=[
            jax.ShapeDtypeStruct((N, C, H, W), jnp.float32),
            jax.ShapeDtypeStruct((N, C, H, W), jnp.float32),
        ],
    )(rows, wsum)


def _sc_body(imgt, ids4, w4, smin, smax, out_rows, out_ws,
             stg_i, stg_w, src_b, idx_b, wv_b, rows_v, scaled, idx_c,
             zb, zw, vb, wb, sm_v, sx_v, band, wband,
             sem_t, sem_g, sem_s, sem_z, sem_w):
    cid = lax.axis_index("c")
    sid = lax.axis_index("s")
    opq = (cid + sid) >> 10
    iota = lax.iota(jnp.int32, 16)
    pad_src = (iota * 401 + (sid * 2 + cid) * 977) % NHW
    pad_idx = (iota * 73 + sid * 131) % R
    zeros16 = jnp.zeros((16,), jnp.float32)

    def _z1(t, _):
        i, j = t // 8, t % 8
        zb[i, pl.ds(j * 16, 16)] = zeros16
        return 0
    lax.fori_loop(0, NB * 8, _z1, 0)

    def _z2(j, _):
        zw[pl.ds(j * 16, 16)] = zeros16
        return 0
    lax.fori_loop(0, TPS // 16, _z2, 0)

    pltpu.async_copy(smin, sm_v, sem_g).wait()
    pltpu.async_copy(smax, sx_v, sem_g).wait()

    def _zero_shared():
        def _zf(j, _):
            pltpu.async_copy(
                zb, band.at[pl.ds(sid * TPS + j * NB, NB)], sem_z)
            return 0

        def _zd(j, _):
            pltpu.make_async_copy(
                zb, band.at[pl.ds(sid * TPS + j * NB, NB)], sem_z).wait()
            return 0

        lax.fori_loop(0, TPS // NB + opq, _zf, 0)
        pltpu.async_copy(zw, wband.at[pl.ds(sid * TPS, TPS)], sem_z)
        lax.fori_loop(0, TPS // NB + opq, _zd, 0)
        pltpu.make_async_copy(
            zw, wband.at[pl.ds(sid * TPS, TPS)], sem_z).wait()

    with jax.named_scope("zero0"):
        _zero_shared()
    plsc.subcore_barrier()

    def _pass(p, _):
        lo = (2 * p + cid) * R
        hi = lo + R

        def _segment(seg, _):
            gs = seg * NSUB + sid
            seg_off = gs * SEG
            smidx = gs + (gs // 56) * 8
            gsv = jnp.full((16,), smidx, jnp.int32)
            sm = plsc.load_gather(sm_v, [gsv])[0]
            sx = plsc.load_gather(sx_v, [gsv])[0]

            @pl.when((sx >= lo) & (sm < hi))
            def _do_segment():
                def _stf(k, _):
                    pltpu.async_copy(ids4.at[k, pl.ds(seg_off, SEG)],
                                     stg_i.at[k], sem_t)
                    pltpu.async_copy(w4.at[k, pl.ds(seg_off, SEG)],
                                     stg_w.at[k], sem_t)
                    return 0

                def _std(k, _):
                    pltpu.make_async_copy(ids4.at[k, pl.ds(seg_off, SEG)],
                                          stg_i.at[k], sem_t).wait()
                    pltpu.make_async_copy(w4.at[k, pl.ds(seg_off, SEG)],
                                          stg_w.at[k], sem_t).wait()
                    return 0

                with jax.named_scope("stage_in"):
                    lax.fori_loop(0, 4 + opq, _stf, 0)
                    lax.fori_loop(0, 4 + opq, _std, 0)

                def _scan(v, count):
                    tids = [stg_i[k, pl.ds(v * 16, 16)] for k in range(4)]
                    inb = [(t >= lo) & (t < hi) for t in tids]
                    m = inb[0] | inb[1] | inb[2] | inb[3]
                    cnt = plsc.all_reduce_population_count(m)[0]
                    svec = iota + (seg_off + v * 16)
                    plsc.store_compressed(src_b.at[pl.ds(count, 16)], svec,
                                          mask=m)
                    for k in range(4):
                        lidx = jnp.clip(tids[k] - lo, 0, R - 1)
                        wk = stg_w[k, pl.ds(v * 16, 16)]
                        weff = jnp.where(inb[k], wk, 0.0)
                        plsc.store_compressed(
                            idx_b.at[pl.ds(k * STRIDE + count, 16)], lidx,
                            mask=m)
                        plsc.store_compressed(
                            wv_b.at[pl.ds(k * STRIDE + count, 16)], weff,
                            mask=m)
                    return count + cnt

                with jax.named_scope("scan"):
                    count = lax.fori_loop(0, NSEGV, _scan, 0)

                target = (count + G - 1) // G * G

                def _pad(i, c):
                    src_b[pl.ds(c, 16)] = pad_src
                    for k in range(4):
                        idx_b[pl.ds(k * STRIDE + c, 16)] = pad_idx
                        wv_b[pl.ds(k * STRIDE + c, 16)] = zeros16
                    return c + 16

                lax.fori_loop(0, (target - count + 15) // 16, _pad, count)

                def _drain_sc():
                    def _d(k, _):
                        pltpu.make_async_copy(
                            scaled.at[k], band.at[idx_c.at[k]],
                            sem_s).wait()
                        pltpu.make_async_copy(
                            wv_b.at[pl.ds(k * STRIDE, G)],
                            wband.at[idx_c.at[k]], sem_s).wait()
                        return 0

                    lax.fori_loop(0, 4 + opq, _d, 0)

                def _chunk(ch, _):
                    c0 = ch * G
                    gat = pltpu.async_copy(
                        imgt.at[src_b.at[pl.ds(c0, G)]], rows_v, sem_g)

                    @pl.when(ch > 0)
                    def _():
                        _drain_sc()

                    for k in range(4):
                        for j in range(G // 16):
                            idx_c[k, pl.ds(j * 16, 16)] = (
                                idx_b[pl.ds(k * STRIDE + c0 + j * 16, 16)])
                    gat.wait()

                    def _row(r, _):
                        rv = [rows_v[r, pl.ds(i * 16, 16)]
                              for i in range(8)]
                        for k in range(4):
                            ridx = jnp.full(
                                (16,), k * STRIDE + c0 + r, jnp.int32)
                            wsc = plsc.load_gather(wv_b, [ridx])
                            for i in range(8):
                                scaled[k, r, pl.ds(i * 16, 16)] = (
                                    rv[i] * wsc)
                        return 0

                    lax.fori_loop(0, G, _row, 0)

                    def _f(k, _):
                        pltpu.async_copy(
                            scaled.at[k], band.at[idx_c.at[k]], sem_s,
                            add=True)
                        pltpu.async_copy(
                            wv_b.at[pl.ds(k * STRIDE + c0, G)],
                            wband.at[idx_c.at[k]], sem_s, add=True)
                        return 0

                    lax.fori_loop(0, 4 + opq, _f, 0)
                    return 0

                with jax.named_scope("chunks"):
                    lax.fori_loop(0, target // G, _chunk, 0)

                @pl.when(target > 0)
                def _():
                    _drain_sc()

            return 0

        lax.fori_loop(0, SHARE // SEG, _segment, 0)
        plsc.subcore_barrier()
        g0 = (2 * p + cid) * R + sid * TPS

        def _wo(j, _):
            b = j % 2

            @pl.when(j >= 2)
            def _():
                pltpu.make_async_copy(
                    vb.at[b], out_rows.at[pl.ds(g0 + j * NB, NB)],
                    sem_w).wait()

            pltpu.async_copy(band.at[pl.ds(sid * TPS + j * NB, NB)],
                             vb.at[b], sem_g).wait()
            pltpu.async_copy(vb.at[b], out_rows.at[pl.ds(g0 + j * NB, NB)],
                             sem_w)
            return 0

        with jax.named_scope("writeout"):
            lax.fori_loop(0, TPS // NB + opq, _wo, 0)

        def _wd(j, _):
            pltpu.make_async_copy(
                vb.at[j], out_rows.at[pl.ds(g0 + j * NB, NB)], sem_w).wait()
            return 0

        lax.fori_loop(0, 2 + opq, _wd, 0)
        pltpu.async_copy(wband.at[pl.ds(sid * TPS, TPS)], wb, sem_g).wait()
        pltpu.async_copy(wb, out_ws.at[pl.ds(g0, TPS)], sem_g).wait()
        with jax.named_scope("rezero"):
            _zero_shared()
        plsc.subcore_barrier()
        return 0

    lax.fori_loop(0, NPASS, _pass, 0)


def _sc_scatter(imgt, ids4, w4, smin, smax):
    kfn = pl.kernel(
        _sc_body,
        out_type=[
            jax.ShapeDtypeStruct((ROWS_PAD, CP), jnp.float32),
            jax.ShapeDtypeStruct((ROWS_PAD,), jnp.float32),
        ],
        mesh=plsc.VectorSubcoreMesh(core_axis_name="c", subcore_axis_name="s"),
        compiler_params=pltpu.CompilerParams(needs_layout_passes=False),
        scratch_types=[
            pltpu.VMEM((4, SEG), jnp.int32),
            pltpu.VMEM((4, SEG), jnp.float32),
            pltpu.VMEM((SEG + G,), jnp.int32),
            pltpu.VMEM((4 * (SEG + G),), jnp.int32),
            pltpu.VMEM((4 * (SEG + G),), jnp.float32),
            pltpu.VMEM((G, CP), jnp.float32),
            pltpu.VMEM((4, G, CP), jnp.float32),
            pltpu.VMEM((4, G), jnp.int32),
            pltpu.VMEM((NB, CP), jnp.float32),
            pltpu.VMEM((TPS,), jnp.float32),
            pltpu.VMEM((2, NB, CP), jnp.float32),
            pltpu.VMEM((TPS,), jnp.float32),
            pltpu.VMEM((256,), jnp.int32),
            pltpu.VMEM((256,), jnp.int32),
            pltpu.VMEM_SHARED((R, CP), jnp.float32),
            pltpu.VMEM_SHARED((R,), jnp.float32),
            pltpu.SemaphoreType.DMA,
            pltpu.SemaphoreType.DMA,
            pltpu.SemaphoreType.DMA,
            pltpu.SemaphoreType.DMA,
            pltpu.SemaphoreType.DMA,
        ],
    )
    return kfn(imgt, ids4, w4, smin, smax)


def kernel(img, flo):
    ids4, w4, smin, smax = _prep(flo)
    ids4 = ids4.reshape(4, NHW)
    w4 = w4.reshape(4, NHW)
    smin = smin[:, 0, :].reshape(256)
    smax = smax[:, 0, :].reshape(256)
    imgt = _to_rows(img)
    out_rows, wsum = _sc_scatter(imgt, ids4, w4, smin, smax)
    return _post(out_rows[:NHW], wsum[:NHW])


# speedup vs baseline: 5.9702x; 1.0026x over previous
import functools

import jax
import jax.numpy as jnp
from jax import lax
from jax.experimental import pallas as pl
from jax.experimental.pallas import tpu as pltpu
from jax.experimental.pallas import tpu_sc as plsc

N, C, H, W = 4, 96, 224, 224
NHW = N * H * W
CP = 128
R = 10240
NPASS = 10
ROWS_PAD = 204800
NSUB = 16
SHARE = NHW // (2 * NSUB) * 2
SEG = 896
NSEGV = SEG // 16
G = 32
STRIDE = SEG + G
TPS = R // NSUB
NB = 16


def _prep_body(flo_ref, ids_ref, w_ref, smin_ref, smax_ref):
    n = pl.program_id(0)
    y = flo_ref[0, 0]
    x = flo_ref[0, 1]
    x1 = jnp.floor(x)
    y1 = jnp.floor(y)
    fx = x - x1
    fy = y - y1
    ih = lax.broadcasted_iota(jnp.int32, (H, W), 0)
    iw = lax.broadcasted_iota(jnp.int32, (H, W), 1)
    ix1 = x1.astype(jnp.int32) + ih
    iy1 = y1.astype(jnp.int32) + iw
    ix2 = ix1 + 1
    iy2 = iy1 + 1
    base = n * (H * W) + ih * W + iw
    a1 = fx * fx
    a2 = (fx - 1.0) * (fx - 1.0)
    b1 = fy * fy
    b2 = (fy - 1.0) * (fy - 1.0)
    mx1 = (ix1 >= 0) & (ix1 < H)
    mx2 = (ix2 >= 0) & (ix2 < H)
    my1 = (iy1 >= 0) & (iy1 < W)
    my2 = (iy2 >= 0) & (iy2 < W)
    combos = (
        (mx1 & my1, ix1, iy1, jnp.exp(-(a1 + b1))),
        (mx1 & my2, ix1, iy2, jnp.exp(-(a1 + b2))),
        (mx2 & my1, ix2, iy1, jnp.exp(-(a2 + b1))),
        (mx2 & my2, ix2, iy2, jnp.exp(-(a2 + b2))),
    )
    tmin = None
    tmax = None
    for k, (m, ix, iy, wk) in enumerate(combos):
        tid = n * (H * W) + ix * W + iy
        tid = jnp.where(m, tid, base)
        ids_ref[k, 0] = tid
        w_ref[k, 0] = jnp.where(m, wk, 0.0)
        tmin = tid if tmin is None else jnp.minimum(tmin, tid)
        tmax = tid if tmax is None else jnp.maximum(tmax, tid)
    sm = jnp.min(jnp.min(tmin.reshape(56, 4, W), axis=1), axis=1)
    sx = jnp.max(jnp.max(tmax.reshape(56, 4, W), axis=1), axis=1)
    smin_ref[0, 0] = jnp.concatenate([sm, jnp.full((8,), 2 ** 30, jnp.int32)])
    smax_ref[0, 0] = jnp.concatenate([sx, jnp.full((8,), -1, jnp.int32)])


def _prep(flo):
    return pl.pallas_call(
        _prep_body,
        grid=(N,),
        in_specs=[pl.BlockSpec((1, 2, H, W), lambda n: (n, 0, 0, 0))],
        out_specs=[
            pl.BlockSpec((4, 1, H, W), lambda n: (0, n, 0, 0)),
            pl.BlockSpec((4, 1, H, W), lambda n: (0, n, 0, 0)),
            pl.BlockSpec((1, 8, 64), lambda n: (n, 0, 0)),
            pl.BlockSpec((1, 8, 64), lambda n: (n, 0, 0)),
        ],
        out_shape=[
            jax.ShapeDtypeStruct((4, N, H, W), jnp.int32),
            jax.ShapeDtypeStruct((4, N, H, W), jnp.float32),
            jax.ShapeDtypeStruct((N, 8, 64), jnp.int32),
            jax.ShapeDtypeStruct((N, 8, 64), jnp.int32),
        ],
    )(flo)


_HC = 7
_HB = H // _HC
_RB = _HB * W


def _tin_body(img_ref, out_ref):
    x = img_ref[0].reshape(C, _RB)
    xp = jnp.concatenate([x, jnp.zeros((CP - C, _RB), jnp.float32)], axis=0)
    out_ref[...] = xp.T


def _to_rows(img):
    return pl.pallas_call(
        _tin_body,
        grid=(N, _HC),
        in_specs=[pl.BlockSpec((1, C, _HB, W), lambda n, h: (n, 0, h, 0))],
        out_specs=pl.BlockSpec((_RB, CP), lambda n, h: (n * _HC + h, 0)),
        out_shape=jax.ShapeDtypeStruct((NHW, CP), jnp.float32),
    )(img)


def _post_body(rows_ref, ws_ref, img_ref, one_ref):
    xt = rows_ref[...].T
    img_ref[0] = xt[:C].reshape(C, _HB, W)
    wv = ws_ref[...]
    one_ref[0] = jnp.broadcast_to(wv[None, :], (C, _RB)).reshape(C, _HB, W)


def _post(rows, wsum):
    return pl.pallas_call(
        _post_body,
        grid=(N, _HC),
        in_specs=[
            pl.BlockSpec((_RB, CP), lambda n, h: (n * _HC + h, 0)),
            pl.BlockSpec((_RB,), lambda n, h: (n * _HC + h,)),
        ],
        out_specs=[
            pl.BlockSpec((1, C, _HB, W), lambda n, h: (n, 0, h, 0)),
            pl.BlockSpec((1, C, _HB, W), lambda n, h: (n, 0, h, 0)),
        ],
        out_shape=[
            jax.ShapeDtypeStruct((N, C, H, W), jnp.float32),
            jax.ShapeDtypeStruct((N, C, H, W), jnp.float32),
        ],
    )(rows, wsum)


def _sc_body(imgt, ids4, w4, smin, smax, out_rows, out_ws,
             stg_i, stg_w, src_b, idx_b, wv_b, rows_v, scaled, idx_c,
             zb, zw, vb, wb, sm_v, sx_v, band, wband,
             sem_t, sem_g, sem_s, sem_z, sem_w):
    cid = lax.axis_index("c")
    sid = lax.axis_index("s")
    opq = (cid + sid) >> 10
    iota = lax.iota(jnp.int32, 16)
    pad_src = (iota * 401 + (sid * 2 + cid) * 977) % NHW
    pad_idx = (iota * 73 + sid * 131) % R
    zeros16 = jnp.zeros((16,), jnp.float32)

    def _z1(t, _):
        i, j = t // 8, t % 8
        zb[i, pl.ds(j * 16, 16)] = zeros16
        return 0
    lax.fori_loop(0, NB * 8, _z1, 0)

    def _z2(j, _):
        zw[pl.ds(j * 16, 16)] = zeros16
        return 0
    lax.fori_loop(0, TPS // 16, _z2, 0)

    pltpu.async_copy(smin, sm_v, sem_g).wait()
    pltpu.async_copy(smax, sx_v, sem_g).wait()

    def _zero_shared():
        def _zf(j, _):
            pltpu.async_copy(
                zb, band.at[pl.ds(sid * TPS + j * NB, NB)], sem_z)
            return 0

        def _zd(j, _):
            pltpu.make_async_copy(
                zb, band.at[pl.ds(sid * TPS + j * NB, NB)], sem_z).wait()
            return 0

        lax.fori_loop(0, TPS // NB + opq, _zf, 0)
        pltpu.async_copy(zw, wband.at[pl.ds(sid * TPS, TPS)], sem_z)
        lax.fori_loop(0, TPS // NB + opq, _zd, 0)
        pltpu.make_async_copy(
            zw, wband.at[pl.ds(sid * TPS, TPS)], sem_z).wait()

    with jax.named_scope("zero0"):
        _zero_shared()
    plsc.subcore_barrier()

    def _pass(p, _):
        lo = (2 * p + cid) * R
        hi = lo + R

        def _segment(seg, _):
            gs = seg * NSUB + sid
            seg_off = gs * SEG
            smidx = gs + (gs // 56) * 8
            gsv = jnp.full((16,), smidx, jnp.int32)
            sm = plsc.load_gather(sm_v, [gsv])[0]
            sx = plsc.load_gather(sx_v, [gsv])[0]

            @pl.when((sx >= lo) & (sm < hi))
            def _do_segment():
                def _stf(k, _):
                    pltpu.async_copy(ids4.at[k, pl.ds(seg_off, SEG)],
                                     stg_i.at[k], sem_t)
                    pltpu.async_copy(w4.at[k, pl.ds(seg_off, SEG)],
                                     stg_w.at[k], sem_t)
                    return 0

                def _std(k, _):
                    pltpu.make_async_copy(ids4.at[k, pl.ds(seg_off, SEG)],
                                          stg_i.at[k], sem_t).wait()
                    pltpu.make_async_copy(w4.at[k, pl.ds(seg_off, SEG)],
                                          stg_w.at[k], sem_t).wait()
                    return 0

                with jax.named_scope("stage_in"):
                    lax.fori_loop(0, 4 + opq, _stf, 0)
                    lax.fori_loop(0, 4 + opq, _std, 0)

                def _scan(v, count):
                    tids = [stg_i[k, pl.ds(v * 16, 16)] for k in range(4)]
                    inb = [(t >= lo) & (t < hi) for t in tids]
                    m = inb[0] | inb[1] | inb[2] | inb[3]
                    cnt = plsc.all_reduce_population_count(m)[0]
                    svec = iota + (seg_off + v * 16)
                    plsc.store_compressed(src_b.at[pl.ds(count, 16)], svec,
                                          mask=m)
                    for k in range(4):
                        lidx = jnp.clip(tids[k] - lo, 0, R - 1)
                        wk = stg_w[k, pl.ds(v * 16, 16)]
                        weff = jnp.where(inb[k], wk, 0.0)
                        plsc.store_compressed(
                            idx_b.at[pl.ds(k * STRIDE + count, 16)], lidx,
                            mask=m)
                        plsc.store_compressed(
                            wv_b.at[pl.ds(k * STRIDE + count, 16)], weff,
                            mask=m)
                    return count + cnt

                with jax.named_scope("scan"):
                    count = lax.fori_loop(0, NSEGV, _scan, 0)

                target = (count + G - 1) // G * G

                def _pad(i, c):
                    src_b[pl.ds(c, 16)] = pad_src
                    for k in range(4):
                        idx_b[pl.ds(k * STRIDE + c, 16)] = pad_idx
                        wv_b[pl.ds(k * STRIDE + c, 16)] = zeros16
                    return c + 16

                lax.fori_loop(0, (target - count + 15) // 16, _pad, count)

                def _drain_sc():
                    def _d(k, _):
                        pltpu.make_async_copy(
                            scaled.at[k], band.at[idx_c.at[k]],
                            sem_s).wait()
                        pltpu.make_async_copy(
                            wv_b.at[pl.ds(k * STRIDE, G)],
                            wband.at[idx_c.at[k]], sem_s).wait()
                        return 0

                    lax.fori_loop(0, 4 + opq, _d, 0)

                nch = target // G

                @pl.when(nch > 0)
                def _():
                    pltpu.async_copy(
                        imgt.at[src_b.at[pl.ds(0, G)]], rows_v, sem_g)

                def _chunk(ch, _):
                    c0 = ch * G

                    @pl.when(ch > 0)
                    def _():
                        _drain_sc()

                    for k in range(4):
                        for j in range(G // 16):
                            idx_c[k, pl.ds(j * 16, 16)] = (
                                idx_b[pl.ds(k * STRIDE + c0 + j * 16, 16)])
                    pltpu.make_async_copy(
                        imgt.at[src_b.at[pl.ds(c0, G)]], rows_v, sem_g).wait()

                    def _row(r, _):
                        rv = [rows_v[r, pl.ds(i * 16, 16)]
                              for i in range(8)]
                        for k in range(4):
                            ridx = jnp.full(
                                (16,), k * STRIDE + c0 + r, jnp.int32)
                            wsc = plsc.load_gather(wv_b, [ridx])
                            for i in range(8):
                                scaled[k, r, pl.ds(i * 16, 16)] = (
                                    rv[i] * wsc)
                        return 0

                    lax.fori_loop(0, G, _row, 0)

                    @pl.when(ch + 1 < nch)
                    def _():
                        pltpu.async_copy(
                            imgt.at[src_b.at[pl.ds(c0 + G, G)]], rows_v,
                            sem_g)

                    def _f(k, _):
                        pltpu.async_copy(
                            scaled.at[k], band.at[idx_c.at[k]], sem_s,
                            add=True)
                        pltpu.async_copy(
                            wv_b.at[pl.ds(k * STRIDE + c0, G)],
                            wband.at[idx_c.at[k]], sem_s, add=True)
                        return 0

                    lax.fori_loop(0, 4 + opq, _f, 0)
                    return 0

                with jax.named_scope("chunks"):
                    lax.fori_loop(0, nch, _chunk, 0)

                @pl.when(target > 0)
                def _():
                    _drain_sc()

            return 0

        lax.fori_loop(0, SHARE // SEG, _segment, 0)
        plsc.subcore_barrier()
        g0 = (2 * p + cid) * R + sid * TPS

        def _wo(j, _):
            b = j % 2

            @pl.when(j >= 2)
            def _():
                pltpu.make_async_copy(
                    vb.at[b], out_rows.at[pl.ds(g0 + j * NB, NB)],
                    sem_w).wait()

            pltpu.async_copy(band.at[pl.ds(sid * TPS + j * NB, NB)],
                             vb.at[b], sem_g).wait()
            pltpu.async_copy(vb.at[b], out_rows.at[pl.ds(g0 + j * NB, NB)],
                             sem_w)
            return 0

        with jax.named_scope("writeout"):
            lax.fori_loop(0, TPS // NB + opq, _wo, 0)

        def _wd(j, _):
            pltpu.make_async_copy(
                vb.at[j], out_rows.at[pl.ds(g0 + j * NB, NB)], sem_w).wait()
            return 0

        lax.fori_loop(0, 2 + opq, _wd, 0)
        pltpu.async_copy(wband.at[pl.ds(sid * TPS, TPS)], wb, sem_g).wait()
        pltpu.async_copy(wb, out_ws.at[pl.ds(g0, TPS)], sem_g).wait()
        with jax.named_scope("rezero"):
            _zero_shared()
        plsc.subcore_barrier()
        return 0

    lax.fori_loop(0, NPASS, _pass, 0)


def _sc_scatter(imgt, ids4, w4, smin, smax):
    kfn = pl.kernel(
        _sc_body,
        out_type=[
            jax.ShapeDtypeStruct((ROWS_PAD, CP), jnp.float32),
            jax.ShapeDtypeStruct((ROWS_PAD,), jnp.float32),
        ],
        mesh=plsc.VectorSubcoreMesh(core_axis_name="c", subcore_axis_name="s"),
        compiler_params=pltpu.CompilerParams(needs_layout_passes=False),
        scratch_types=[
            pltpu.VMEM((4, SEG), jnp.int32),
            pltpu.VMEM((4, SEG), jnp.float32),
            pltpu.VMEM((SEG + G,), jnp.int32),
            pltpu.VMEM((4 * (SEG + G),), jnp.int32),
            pltpu.VMEM((4 * (SEG + G),), jnp.float32),
            pltpu.VMEM((G, CP), jnp.float32),
            pltpu.VMEM((4, G, CP), jnp.float32),
            pltpu.VMEM((4, G), jnp.int32),
            pltpu.VMEM((NB, CP), jnp.float32),
            pltpu.VMEM((TPS,), jnp.float32),
            pltpu.VMEM((2, NB, CP), jnp.float32),
            pltpu.VMEM((TPS,), jnp.float32),
            pltpu.VMEM((256,), jnp.int32),
            pltpu.VMEM((256,), jnp.int32),
            pltpu.VMEM_SHARED((R, CP), jnp.float32),
            pltpu.VMEM_SHARED((R,), jnp.float32),
            pltpu.SemaphoreType.DMA,
            pltpu.SemaphoreType.DMA,
            pltpu.SemaphoreType.DMA,
            pltpu.SemaphoreType.DMA,
            pltpu.SemaphoreType.DMA,
        ],
    )
    return kfn(imgt, ids4, w4, smin, smax)


def kernel(img, flo):
    ids4, w4, smin, smax = _prep(flo)
    ids4 = ids4.reshape(4, NHW)
    w4 = w4.reshape(4, NHW)
    smin = smin[:, 0, :].reshape(256)
    smax = smax[:, 0, :].reshape(256)
    imgt = _to_rows(img)
    out_rows, wsum = _sc_scatter(imgt, ids4, w4, smin, smax)
    return _post(out_rows[:NHW], wsum[:NHW])
